# baseline (speedup 1.0000x reference)
import jax
import jax.numpy as jnp
import numpy as np
from jax import lax
from jax.experimental import pallas as pl
from jax.experimental.pallas import tpu as pltpu

_B = 16384
_BIT = 128
_ROWS = 2048

_K0 = np.uint32(0)
_K1 = np.uint32(42)
_KS2 = np.uint32(int(_K0) ^ int(_K1) ^ 0x1BD11BDA)
_ROT_A = (13, 15, 26, 6)
_ROT_B = (17, 29, 16, 24)


def _rotl(v, r):
    return (v << jnp.uint32(r)) | (v >> jnp.uint32(32 - r))


def _threefry_mix(x1):
    ks = (_K0, _K1, _KS2)
    rots = (_ROT_A, _ROT_B)
    x0 = x1
    x1 = _rotl(x1, rots[0][0]) ^ x0
    for r in rots[0][1:]:
        x0 = x0 + x1
        x1 = _rotl(x1, r)
        x1 = x1 ^ x0
    x0 = x0 + ks[1]
    x1 = x1 + ks[2] + jnp.uint32(1)
    for i in range(1, 5):
        for r in rots[i % 2]:
            x0 = x0 + x1
            x1 = _rotl(x1, r)
            x1 = x1 ^ x0
        x0 = x0 + ks[(i + 1) % 3]
        x1 = x1 + ks[(i + 2) % 3] + jnp.uint32(i + 1)
    return x0 ^ x1


def _bits_kernel(x_ref, o_ref, iota_ref):
    i = pl.program_id(0)

    @pl.when(i == 0)
    def _init():
        row = lax.broadcasted_iota(jnp.uint32, (_ROWS, _BIT), 0)
        col = lax.broadcasted_iota(jnp.uint32, (_ROWS, _BIT), 1)
        iota_ref[...] = row * jnp.uint32(_BIT) + col + jnp.uint32(int(_K1))

    base = jnp.uint32(i * (_ROWS * _BIT))
    x1 = base + iota_ref[...]
    bits = _threefry_mix(x1)
    m = bits >> jnp.uint32(9)
    p = x_ref[...]
    t = jnp.ceil(p * jnp.float32(8388608.0)).astype(jnp.uint32)
    o_ref[...] = jnp.where(m < t, 1.0, 0.0).astype(jnp.float32)


def kernel(x):
    out = pl.pallas_call(
        _bits_kernel,
        grid=(_B // _ROWS,),
        in_specs=[pl.BlockSpec((_ROWS, 1), lambda i: (i, 0))],
        out_specs=pl.BlockSpec((_ROWS, _BIT), lambda i: (i, 0)),
        out_shape=jax.ShapeDtypeStruct((_B, _BIT), jnp.float32),
        scratch_shapes=[pltpu.VMEM((_ROWS, _BIT), jnp.uint32)],
        compiler_params=pltpu.CompilerParams(
            dimension_semantics=("arbitrary",),
        ),
    )(x.reshape(_B, 1))
    return out


# speedup vs baseline: 1.6584x; 1.6584x over previous
import jax
import jax.numpy as jnp
import numpy as np
from jax import lax
from jax.experimental import pallas as pl
from jax.experimental.pallas import tpu as pltpu

_B = 16384
_BIT = 128
_ROWS = 2048

_K0 = np.uint32(0)
_K1 = np.uint32(42)
_KS2 = np.uint32(int(_K0) ^ int(_K1) ^ 0x1BD11BDA)
_ROT_A = (13, 15, 26, 6)
_ROT_B = (17, 29, 16, 24)


def _rotl(v, r):
    return (v << jnp.uint32(r)) | (v >> jnp.uint32(32 - r))


def _threefry_mix(x1):
    ks = (_K0, _K1, _KS2)
    rots = (_ROT_A, _ROT_B)
    x0 = x1
    x1 = _rotl(x1, rots[0][0]) ^ x0
    for r in rots[0][1:]:
        x0 = x0 + x1
        x1 = _rotl(x1, r)
        x1 = x1 ^ x0
    x0 = x0 + ks[1]
    x1 = x1 + ks[2] + jnp.uint32(1)
    for i in range(1, 5):
        for r in rots[i % 2]:
            x0 = x0 + x1
            x1 = _rotl(x1, r)
            x1 = x1 ^ x0
        x0 = x0 + ks[(i + 1) % 3]
        x1 = x1 + ks[(i + 2) % 3] + jnp.uint32(i + 1)
    return x0 ^ x1


def _bits_kernel(x_ref, o_ref, iota_ref):
    i = pl.program_id(0)

    @pl.when(i == 0)
    def _init():
        row = lax.broadcasted_iota(jnp.uint32, (_ROWS, _BIT), 0)
        col = lax.broadcasted_iota(jnp.uint32, (_ROWS, _BIT), 1)
        iota_ref[...] = row * jnp.uint32(_BIT) + col + jnp.uint32(int(_K1))

    base = jnp.uint32(i * (_ROWS * _BIT))
    x1 = base + iota_ref[...]
    bits = _threefry_mix(x1)
    m = bits >> jnp.uint32(9)
    mf = lax.bitcast_convert_type(m, jnp.int32).astype(jnp.float32)
    p = x_ref[...]
    t = p * jnp.float32(8388608.0)
    o_ref[...] = jnp.where(mf < t, 1.0, 0.0).astype(jnp.float32)


def kernel(x):
    out = pl.pallas_call(
        _bits_kernel,
        grid=(_B // _ROWS,),
        in_specs=[pl.BlockSpec((_ROWS, 1), lambda i: (i, 0))],
        out_specs=pl.BlockSpec((_ROWS, _BIT), lambda i: (i, 0)),
        out_shape=jax.ShapeDtypeStruct((_B, _BIT), jnp.float32),
        scratch_shapes=[pltpu.VMEM((_ROWS, _BIT), jnp.uint32)],
        compiler_params=pltpu.CompilerParams(
            dimension_semantics=("arbitrary",),
        ),
    )(x.reshape(_B, 1))
    return out


# speedup vs baseline: 1.6605x; 1.0013x over previous
import jax
import jax.numpy as jnp
import numpy as np
from jax import lax
from jax.experimental import pallas as pl
from jax.experimental.pallas import tpu as pltpu

_B = 16384
_BIT = 128
_ROWS = 2048

_K0 = np.uint32(0)
_K1 = np.uint32(42)
_KS2 = np.uint32(int(_K0) ^ int(_K1) ^ 0x1BD11BDA)
_ROT_A = (13, 15, 26, 6)
_ROT_B = (17, 29, 16, 24)


def _rotl(v, r):
    return (v << jnp.uint32(r)) | (v >> jnp.uint32(32 - r))


def _threefry_mix(x1):
    ks = (_K0, _K1, _KS2)
    rots = (_ROT_A, _ROT_B)
    x0 = x1
    x1 = _rotl(x1, rots[0][0]) ^ x0
    for r in rots[0][1:]:
        x0 = x0 + x1
        x1 = _rotl(x1, r)
        x1 = x1 ^ x0
    x0 = x0 + ks[1]
    x1 = x1 + ks[2] + jnp.uint32(1)
    for i in range(1, 5):
        for r in rots[i % 2]:
            x0 = x0 + x1
            x1 = _rotl(x1, r)
            x1 = x1 ^ x0
        if int(ks[(i + 1) % 3]) != 0:
            x0 = x0 + ks[(i + 1) % 3]
        x1 = x1 + ks[(i + 2) % 3] + jnp.uint32(i + 1)
    return x0 ^ x1


def _bits_kernel(x_ref, o_ref, iota_ref):
    i = pl.program_id(0)

    @pl.when(i == 0)
    def _init():
        row = lax.broadcasted_iota(jnp.uint32, (_ROWS, _BIT), 0)
        col = lax.broadcasted_iota(jnp.uint32, (_ROWS, _BIT), 1)
        iota_ref[...] = row * jnp.uint32(_BIT) + col + jnp.uint32(int(_K1))

    base = jnp.uint32(i * (_ROWS * _BIT))
    x1 = base + iota_ref[...]
    bits = _threefry_mix(x1)
    m = bits >> jnp.uint32(9)
    mf = lax.bitcast_convert_type(m, jnp.int32).astype(jnp.float32)
    p = x_ref[...]
    t = p * jnp.float32(8388608.0)
    o_ref[...] = jnp.where(mf < t, 1.0, 0.0).astype(jnp.float32)


def kernel(x):
    out = pl.pallas_call(
        _bits_kernel,
        grid=(_B // _ROWS,),
        in_specs=[pl.BlockSpec((_ROWS, 1), lambda i: (i, 0))],
        out_specs=pl.BlockSpec((_ROWS, _BIT), lambda i: (i, 0)),
        out_shape=jax.ShapeDtypeStruct((_B, _BIT), jnp.float32),
        scratch_shapes=[pltpu.VMEM((_ROWS, _BIT), jnp.uint32)],
        compiler_params=pltpu.CompilerParams(
            dimension_semantics=("parallel",),
        ),
    )(x.reshape(_B, 1))
    return out


# speedup vs baseline: 1.6851x; 1.0148x over previous
import jax
import jax.numpy as jnp
import numpy as np
from jax import lax
from jax.experimental import pallas as pl
from jax.experimental.pallas import tpu as pltpu

_B = 16384
_BIT = 128
_ROWS = 2048

_K0 = np.uint32(0)
_K1 = np.uint32(42)
_KS2 = np.uint32(int(_K0) ^ int(_K1) ^ 0x1BD11BDA)
_ROT_A = (13, 15, 26, 6)
_ROT_B = (17, 29, 16, 24)


def _rotl(v, r):
    return (v << jnp.uint32(r)) | (v >> jnp.uint32(32 - r))


def _threefry_mix(x1):
    ks = (_K0, _K1, _KS2)
    rots = (_ROT_A, _ROT_B)
    x0 = x1
    x1 = _rotl(x1, rots[0][0]) ^ x0
    for r in rots[0][1:]:
        x0 = x0 + x1
        x1 = _rotl(x1, r)
        x1 = x1 ^ x0
    x0 = x0 + ks[1]
    x1 = x1 + ks[2] + jnp.uint32(1)
    for i in range(1, 5):
        for r in rots[i % 2]:
            x0 = x0 + x1
            x1 = _rotl(x1, r)
            x1 = x1 ^ x0
        if int(ks[(i + 1) % 3]) != 0:
            x0 = x0 + ks[(i + 1) % 3]
        x1 = x1 + ks[(i + 2) % 3] + jnp.uint32(i + 1)
    return x0 ^ x1


def _bits_kernel(x_ref, o_ref, iota_ref):
    i = pl.program_id(0)

    @pl.when(i == 0)
    def _init():
        row = lax.broadcasted_iota(jnp.uint32, (_ROWS, _BIT), 0)
        col = lax.broadcasted_iota(jnp.uint32, (_ROWS, _BIT), 1)
        iota_ref[...] = row * jnp.uint32(_BIT) + col + jnp.uint32(int(_K1))

    base = jnp.uint32(i * (_ROWS * _BIT))
    x1 = base + iota_ref[...]
    bits = _threefry_mix(x1)
    t9 = x_ref[...]
    o_ref[...] = jnp.where(bits < t9, 1.0, 0.0).astype(jnp.float32)


def kernel(x):
    t = jnp.minimum(jnp.ceil(x * jnp.float32(8388608.0)),
                    jnp.float32(8388607.0)).astype(jnp.uint32)
    t9 = (t << jnp.uint32(9)).reshape(_B, 1)
    out = pl.pallas_call(
        _bits_kernel,
        grid=(_B // _ROWS,),
        in_specs=[pl.BlockSpec((_ROWS, 1), lambda i: (i, 0))],
        out_specs=pl.BlockSpec((_ROWS, _BIT), lambda i: (i, 0)),
        out_shape=jax.ShapeDtypeStruct((_B, _BIT), jnp.float32),
        scratch_shapes=[pltpu.VMEM((_ROWS, _BIT), jnp.uint32)],
        compiler_params=pltpu.CompilerParams(
            dimension_semantics=("parallel",),
        ),
    )(t9)
    return out


# speedup vs baseline: 1.6859x; 1.0004x over previous
import jax
import jax.numpy as jnp
import numpy as np
from jax import lax
from jax.experimental import pallas as pl
from jax.experimental.pallas import tpu as pltpu

_B = 16384
_BIT = 128
_ROWS = 1024

_K0 = np.uint32(0)
_K1 = np.uint32(42)
_KS2 = np.uint32(int(_K0) ^ int(_K1) ^ 0x1BD11BDA)
_ROT_A = (13, 15, 26, 6)
_ROT_B = (17, 29, 16, 24)


def _rotl(v, r):
    return (v << jnp.uint32(r)) | (v >> jnp.uint32(32 - r))


def _threefry_mix(x1):
    ks = (_K0, _K1, _KS2)
    rots = (_ROT_A, _ROT_B)
    x0 = x1
    x1 = _rotl(x1, rots[0][0]) ^ x0
    for r in rots[0][1:]:
        x0 = x0 + x1
        x1 = _rotl(x1, r)
        x1 = x1 ^ x0
    x0 = x0 + ks[1]
    x1 = x1 + ks[2] + jnp.uint32(1)
    for i in range(1, 5):
        for r in rots[i % 2]:
            x0 = x0 + x1
            x1 = _rotl(x1, r)
            x1 = x1 ^ x0
        if int(ks[(i + 1) % 3]) != 0:
            x0 = x0 + ks[(i + 1) % 3]
        x1 = x1 + ks[(i + 2) % 3] + jnp.uint32(i + 1)
    return x0 ^ x1


def _bits_kernel(x_ref, o_ref, iota_ref):
    i = pl.program_id(0)

    @pl.when(i == 0)
    def _init():
        row = lax.broadcasted_iota(jnp.uint32, (_ROWS, _BIT), 0)
        col = lax.broadcasted_iota(jnp.uint32, (_ROWS, _BIT), 1)
        iota_ref[...] = row * jnp.uint32(_BIT) + col + jnp.uint32(int(_K1))

    base = jnp.uint32(i * (_ROWS * _BIT))
    x1 = base + iota_ref[...]
    bits = _threefry_mix(x1)
    t9 = x_ref[...]
    o_ref[...] = jnp.where(bits < t9, 1.0, 0.0).astype(jnp.float32)


def kernel(x):
    t = jnp.minimum(jnp.ceil(x * jnp.float32(8388608.0)),
                    jnp.float32(8388607.0)).astype(jnp.uint32)
    t9 = (t << jnp.uint32(9)).reshape(_B, 1)
    out = pl.pallas_call(
        _bits_kernel,
        grid=(_B // _ROWS,),
        in_specs=[pl.BlockSpec((_ROWS, 1), lambda i: (i, 0))],
        out_specs=pl.BlockSpec((_ROWS, _BIT), lambda i: (i, 0)),
        out_shape=jax.ShapeDtypeStruct((_B, _BIT), jnp.float32),
        scratch_shapes=[pltpu.VMEM((_ROWS, _BIT), jnp.uint32)],
        compiler_params=pltpu.CompilerParams(
            dimension_semantics=("parallel",),
        ),
    )(t9)
    return out
